# baseline (speedup 1.0000x reference)
import functools

import jax
import jax.numpy as jnp
from jax import lax
from jax.experimental import pallas as pl
from jax.experimental.pallas import tpu as pltpu
from jax.experimental.pallas import tpu_sc as plsc


@functools.lru_cache(maxsize=None)
def _gather_call(b, l, d):
    info = plsc.get_sparse_core_info()
    nc, ns, nl = info.num_cores, info.num_subcores, info.num_lanes
    nw = nc * ns
    bw = b // nw
    assert b % nw == 0 and l % 2 == 0 and d == 64 and bw % nl == 0
    nj = bw // nl

    mesh = plsc.VectorSubcoreMesh(core_axis_name="c", subcore_axis_name="s")

    @functools.partial(
        pl.kernel,
        out_type=jax.ShapeDtypeStruct((l, d, b), jnp.float32),
        mesh=mesh,
        scratch_types=[
            pltpu.VMEM((l, bw), jnp.int32),
            pltpu.VMEM((l, bw), jnp.int32),
            pltpu.VMEM((2, bw, 2 * d), jnp.float32),
            pltpu.VMEM((2, d, bw), jnp.float32),
            pltpu.SemaphoreType.DMA,
            pltpu.SemaphoreType.DMA,
            pltpu.SemaphoreType.DMA,
            pltpu.SemaphoreType.DMA,
        ],
        compiler_params=pltpu.CompilerParams(
            use_tc_tiling_on_sc=False, skip_device_barrier=True,
            needs_layout_passes=False),
    )
    def k(idx_hbm, table_hbm, out_hbm, idx2_v, par_v, wide_v, plane_v,
          sg0, sg1, so0, so1):
        wid = lax.axis_index("s") * nc + lax.axis_index("c")
        base = wid * bw
        pltpu.sync_copy(idx_hbm.at[:, pl.ds(base, bw)], idx2_v)

        def prep(row, carry):
            for j in range(nj):
                v = idx2_v[row, pl.ds(j * nl, nl)]
                par_v[row, pl.ds(j * nl, nl)] = (v & 1) * 64
                idx2_v[row, pl.ds(j * nl, nl)] = lax.shift_right_logical(v, 1)
            return carry

        lax.fori_loop(0, l, prep, 0)

        sg = (sg0, sg1)
        so = (so0, so1)
        jot = [lax.iota(jnp.int32, nl) + j * nl for j in range(nj)]

        def issue_gather(row, s):
            pltpu.async_copy(table_hbm.at[idx2_v.at[row]], wide_v.at[s], sg[s])

        def wait_gather(s):
            pltpu.make_async_copy(
                table_hbm.at[idx2_v.at[0]], wide_v.at[s], sg[s]).wait()

        def transpose(row, s):
            for j in range(nj):
                pv = par_v[row, pl.ds(j * nl, nl)]
                for dd in range(d):
                    val = plsc.load_gather(wide_v.at[s], [jot[j], pv + dd])
                    plane_v[s, dd, pl.ds(j * nl, nl)] = val

        def issue_store(row, s):
            pltpu.async_copy(
                plane_v.at[s], out_hbm.at[row, :, pl.ds(base, bw)], so[s])

        def wait_store(s):
            pltpu.make_async_copy(
                plane_v.at[s], out_hbm.at[0, :, pl.ds(base, bw)], so[s]).wait()

        issue_gather(0, 0)

        def pair(p, carry):
            r0 = 2 * p

            @pl.when(r0 + 1 < l)
            def _():
                issue_gather(r0 + 1, 1)

            wait_gather(0)

            @pl.when(p > 0)
            def _():
                wait_store(0)

            transpose(r0, 0)
            issue_store(r0, 0)

            @pl.when(r0 + 2 < l)
            def _():
                issue_gather(r0 + 2, 0)

            wait_gather(1)

            @pl.when(p > 0)
            def _():
                wait_store(1)

            transpose(r0 + 1, 1)
            issue_store(r0 + 1, 1)
            return carry

        lax.fori_loop(0, l // 2, pair, 0)
        wait_store(0)
        wait_store(1)

    return k


def kernel(inputs, table):
    b, l = inputs.shape
    v, d = table.shape
    idx_t = inputs.T.astype(jnp.int32)
    table2 = table.reshape(v // 2, 2 * d)
    out_t = _gather_call(b, l, d)(idx_t, table2)
    return out_t.transpose(2, 0, 1)


# speedup vs baseline: 1.7336x; 1.7336x over previous
import functools

import jax
import jax.numpy as jnp
from jax import lax
from jax.experimental import pallas as pl
from jax.experimental.pallas import tpu as pltpu
from jax.experimental.pallas import tpu_sc as plsc

_G = 4


@functools.lru_cache(maxsize=None)
def _gather_call(b, l, d):
    info = plsc.get_sparse_core_info()
    nc, ns = info.num_cores, info.num_subcores
    nw = nc * ns
    b_per_w = b // nw
    n_chunks = b_per_w // _G
    assert b % nw == 0 and b_per_w % _G == 0 and n_chunks % 2 == 0

    mesh = plsc.VectorSubcoreMesh(core_axis_name="c", subcore_axis_name="s")

    @functools.partial(
        pl.kernel,
        out_type=jax.ShapeDtypeStruct((b, l, d), jnp.float32),
        mesh=mesh,
        scratch_types=[
            pltpu.VMEM((b_per_w, l), jnp.int32),
            pltpu.VMEM((2, _G, l, d), jnp.float32),
            pltpu.SemaphoreType.DMA,
            pltpu.SemaphoreType.DMA,
            pltpu.SemaphoreType.DMA,
            pltpu.SemaphoreType.DMA,
        ],
        compiler_params=pltpu.CompilerParams(
            use_tc_tiling_on_sc=False, skip_device_barrier=True),
    )
    def k(idx_hbm, table_hbm, out_hbm, idx_v, rows_v, sg0, sg1, so0, so1):
        wid = lax.axis_index("s") * nc + lax.axis_index("c")
        base = wid * b_per_w
        pltpu.sync_copy(idx_hbm.at[pl.ds(base, b_per_w)], idx_v)

        sg = (sg0, sg1)
        so = (so0, so1)

        def issue_gather(g, s):
            for j in range(_G):
                pltpu.async_copy(
                    table_hbm.at[idx_v.at[g * _G + j]], rows_v.at[s, j], sg[s])

        def wait_gather(s):
            for j in range(_G):
                pltpu.make_async_copy(
                    table_hbm.at[idx_v.at[j]], rows_v.at[s, j], sg[s]).wait()

        def issue_store(g, s):
            pltpu.async_copy(
                rows_v.at[s], out_hbm.at[pl.ds(base + g * _G, _G)], so[s])

        def wait_store(s):
            pltpu.make_async_copy(
                rows_v.at[s], out_hbm.at[pl.ds(base, _G)], so[s]).wait()

        issue_gather(0, 0)

        def pair(p, carry):
            g0 = 2 * p

            @pl.when(p > 0)
            def _():
                wait_store(1)

            issue_gather(g0 + 1, 1)
            wait_gather(0)
            issue_store(g0, 0)

            @pl.when(p + 1 < n_chunks // 2)
            def _():
                wait_store(0)
                issue_gather(g0 + 2, 0)

            wait_gather(1)
            issue_store(g0 + 1, 1)
            return carry

        lax.fori_loop(0, n_chunks // 2, pair, 0)
        wait_store(0)
        wait_store(1)

    return k


def kernel(inputs, table):
    b, l = inputs.shape
    return _gather_call(b, l, table.shape[1])(inputs.astype(jnp.int32), table)
